# Initial kernel scaffold; baseline (speedup 1.0000x reference)
import jax
import jax.numpy as jnp
from jax.experimental import pallas as pl


def kernel(x, token_types, token_table, segment_table, pe):
    raise NotImplementedError("write your pallas kernel here")


# speedup vs baseline: 1.1404x; 1.1404x over previous
import functools

import jax
import jax.numpy as jnp
from jax import lax
from jax.experimental import pallas as pl
from jax.experimental.pallas import tpu as pltpu
from jax.experimental.pallas import tpu_sc as plsc

D = 64
L = 16
NW = 32
SEQ = 200
BATCH = 1024
N = BATCH * SEQ
CHUNK = 128
ROWS_PER_W = N // NW
NCH = ROWS_PER_W // CHUNK


def _sc_body(xi_hbm, tt_hbm, table_hbm, seg_hbm, pe_hbm, out_hbm,
             xi_v, tt_v, pe_v, seg_v, add_v, rows_v, sem_rows):
    wid = lax.axis_index("s") * 2 + lax.axis_index("c")
    crow = wid * NCH

    pltpu.sync_copy(xi_hbm.at[wid], xi_v)
    pltpu.sync_copy(tt_hbm.at[wid], tt_v)
    pltpu.sync_copy(pe_hbm.at[pl.ds(0, SEQ)], pe_v)
    pltpu.sync_copy(seg_hbm, seg_v)

    seg0 = [seg_v[0, pl.ds(d * L, L)] for d in range(4)]
    seg1 = [seg_v[1, pl.ds(d * L, L)] for d in range(4)]

    def build_add(s, _):
        for d in range(4):
            p = pe_v[s, pl.ds(d * L, L)]
            add_v[s, pl.ds(d * L, L)] = p + seg0[d]
            add_v[SEQ + s, pl.ds(d * L, L)] = p + seg1[d]
        return 0

    lax.fori_loop(0, SEQ, build_add, 0)

    def chunk_body(c, _):
        pltpu.async_copy(table_hbm.at[xi_v.at[c]], rows_v, sem_rows).wait()

        def add_body(g, _):
            tvec = tt_v[c, pl.ds(g * L, L)]
            base = c * CHUNK + g * L
            for r16 in range(L):
                r = g * L + r16
                row = tvec[r16] * SEQ + lax.rem(base + r16, SEQ)
                for d in range(4):
                    sl = pl.ds(d * L, L)
                    rows_v[r, sl] = rows_v[r, sl] + add_v[row, sl]
            return 0

        lax.fori_loop(0, CHUNK // L, add_body, 0)
        pltpu.sync_copy(rows_v, out_hbm.at[pl.ds((crow + c) * CHUNK, CHUNK)])
        return 0

    lax.fori_loop(0, NCH, chunk_body, 0)


@jax.jit
def _encode(xi, tt, token_table, segment_table, pe2d):
    mesh = plsc.VectorSubcoreMesh(
        core_axis_name="c", subcore_axis_name="s", num_cores=2, num_subcores=16)
    run = pl.kernel(
        _sc_body,
        out_type=jax.ShapeDtypeStruct((N, D), jnp.float32),
        mesh=mesh,
        compiler_params=pltpu.CompilerParams(use_tc_tiling_on_sc=False),
        scratch_types=[
            pltpu.VMEM((NCH, CHUNK), jnp.int32),
            pltpu.VMEM((NCH, CHUNK), jnp.int32),
            pltpu.VMEM((SEQ, D), jnp.float32),
            pltpu.VMEM((2, D), jnp.float32),
            pltpu.VMEM((2 * SEQ, D), jnp.float32),
            pltpu.VMEM((CHUNK, D), jnp.float32),
            pltpu.SemaphoreType.DMA,
        ],
    )
    return run(xi, tt, token_table, segment_table, pe2d)


def kernel(x, token_types, token_table, segment_table, pe):
    xi = x.reshape(N).astype(jnp.int32).reshape(NW, NCH, CHUNK)
    tt = token_types.reshape(N).astype(jnp.int32).reshape(NW, NCH, CHUNK)
    pe2d = pe.reshape(pe.shape[-2], D)
    out = _encode(xi, tt, token_table, segment_table, pe2d)
    return out.reshape(BATCH, SEQ, D)


# speedup vs baseline: 1.1661x; 1.0226x over previous
import functools

import jax
import jax.numpy as jnp
from jax import lax
from jax.experimental import pallas as pl
from jax.experimental.pallas import tpu as pltpu
from jax.experimental.pallas import tpu_sc as plsc

D = 64
L = 16
NW = 32
SEQ = 200
BATCH = 1024
N = BATCH * SEQ
CHUNK = 128
ROWS_PER_W = N // NW
NCH = ROWS_PER_W // CHUNK


def _sc_body(xi_hbm, tt_hbm, table_hbm, seg_hbm, pe_hbm, out_hbm,
             xi_v, tt_v, pe_v, seg_v, add_v, pidx_v, pair_v, out_v, sem_rows):
    wid = lax.axis_index("s") * 2 + lax.axis_index("c")
    crow = wid * NCH

    pltpu.sync_copy(xi_hbm.at[wid], xi_v)
    pltpu.sync_copy(tt_hbm.at[wid], tt_v)
    pltpu.sync_copy(pe_hbm.at[pl.ds(0, SEQ)], pe_v)
    pltpu.sync_copy(seg_hbm, seg_v)

    seg0 = [seg_v[0, pl.ds(d * L, L)] for d in range(4)]
    seg1 = [seg_v[1, pl.ds(d * L, L)] for d in range(4)]

    def build_add(s, _):
        for d in range(4):
            p = pe_v[s, pl.ds(d * L, L)]
            add_v[s, pl.ds(d * L, L)] = p + seg0[d]
            add_v[SEQ + s, pl.ds(d * L, L)] = p + seg1[d]
        return 0

    lax.fori_loop(0, SEQ, build_add, 0)

    def chunk_body(c, _):
        def build_pidx(g, _):
            pidx_v[pl.ds(g * L, L)] = lax.shift_right_logical(
                xi_v[c, pl.ds(g * L, L)], 1)
            return 0

        lax.fori_loop(0, CHUNK // L, build_pidx, 0)
        pltpu.async_copy(table_hbm.at[pidx_v], pair_v, sem_rows).wait()

        def add_body(g, _):
            tvec = tt_v[c, pl.ds(g * L, L)]
            hvec = xi_v[c, pl.ds(g * L, L)]
            base = c * CHUNK + g * L
            for r16 in range(L):
                r = g * L + r16
                row = tvec[r16] * SEQ + lax.rem(base + r16, SEQ)
                half = (hvec[r16] & 1) * D
                for d in range(4):
                    sl = pl.ds(d * L, L)
                    out_v[r, sl] = (pair_v[r, pl.ds(half + d * L, L)]
                                    + add_v[row, sl])
            return 0

        lax.fori_loop(0, CHUNK // L, add_body, 0)
        pltpu.sync_copy(out_v, out_hbm.at[pl.ds((crow + c) * CHUNK, CHUNK)])
        return 0

    lax.fori_loop(0, NCH, chunk_body, 0)


@jax.jit
def _encode(xi, tt, table2, segment_table, pe2d):
    mesh = plsc.VectorSubcoreMesh(
        core_axis_name="c", subcore_axis_name="s", num_cores=2, num_subcores=16)
    run = pl.kernel(
        _sc_body,
        out_type=jax.ShapeDtypeStruct((N, D), jnp.float32),
        mesh=mesh,
        scratch_types=[
            pltpu.VMEM((NCH, CHUNK), jnp.int32),
            pltpu.VMEM((NCH, CHUNK), jnp.int32),
            pltpu.VMEM((SEQ, D), jnp.float32),
            pltpu.VMEM((2, D), jnp.float32),
            pltpu.VMEM((2 * SEQ, D), jnp.float32),
            pltpu.VMEM((CHUNK,), jnp.int32),
            pltpu.VMEM((CHUNK, 2 * D), jnp.float32),
            pltpu.VMEM((CHUNK, D), jnp.float32),
            pltpu.SemaphoreType.DMA,
        ],
    )
    return run(xi, tt, table2, segment_table, pe2d)


def kernel(x, token_types, token_table, segment_table, pe):
    xi = x.reshape(N).astype(jnp.int32).reshape(NW, NCH, CHUNK)
    tt = token_types.reshape(N).astype(jnp.int32).reshape(NW, NCH, CHUNK)
    table2 = token_table.reshape(token_table.shape[0] // 2, 2 * D)
    pe2d = pe.reshape(pe.shape[-2], D)
    out = _encode(xi, tt, table2, segment_table, pe2d)
    return out.reshape(BATCH, SEQ, D)
